# baseline (speedup 1.0000x reference)
import functools

import jax
import jax.numpy as jnp
from jax import lax
from jax.experimental import pallas as pl
from jax.experimental.pallas import tpu as pltpu
from jax.experimental.pallas import tpu_sc as plsc

VOCAB = 1000
B_TOT = 1024 * 50
NC, NS = 2, 16
NW = NC * NS
BPW = B_TOT // NW
CH = 40
NBUF = 3
NCH = BPW // CH


@functools.partial(
    pl.kernel,
    out_type=jax.ShapeDtypeStruct((B_TOT, VOCAB), jnp.float32),
    mesh=plsc.VectorSubcoreMesh(core_axis_name="c", subcore_axis_name="s"),
    scratch_types=[
        pltpu.VMEM((BPW,), jnp.int32),
        pltpu.VMEM((NBUF, CH, VOCAB), jnp.float32),
        pltpu.SemaphoreType.DMA,
        pltpu.SemaphoreType.DMA,
    ],
    compiler_params=pltpu.CompilerParams(use_tc_tiling_on_sc=False),
)
def _sc_gather(idx_hbm, table_hbm, out_hbm, idx_v, rows_v, sem_g, sem_s):
    wid = lax.axis_index("s") * NC + lax.axis_index("c")
    base = wid * BPW
    pltpu.sync_copy(idx_hbm.at[pl.ds(base, BPW)], idx_v)

    gathers = [None] * NCH
    stores = [None] * NCH
    for c in range(NCH):
        if c >= NBUF:
            stores[c - NBUF].wait()
        gathers[c] = pltpu.async_copy(
            table_hbm.at[idx_v.at[pl.ds(c * CH, CH)]], rows_v.at[c % NBUF], sem_g
        )
        if c >= 1:
            gathers[c - 1].wait()
            stores[c - 1] = pltpu.async_copy(
                rows_v.at[(c - 1) % NBUF],
                out_hbm.at[pl.ds(base + (c - 1) * CH, CH)],
                sem_s,
            )
    gathers[NCH - 1].wait()
    stores[NCH - 1] = pltpu.async_copy(
        rows_v.at[(NCH - 1) % NBUF],
        out_hbm.at[pl.ds(base + (NCH - 1) * CH, CH)],
        sem_s,
    )
    for c in range(NCH - NBUF, NCH):
        stores[c].wait()


def kernel(idx, table):
    b, s = idx.shape
    idx_flat = idx.reshape(-1).astype(jnp.int32)
    out = _sc_gather(idx_flat, table)
    return out.reshape(b, s, VOCAB)


# speedup vs baseline: 1.0006x; 1.0006x over previous
import functools

import jax
import jax.numpy as jnp
from jax import lax
from jax.experimental import pallas as pl
from jax.experimental.pallas import tpu as pltpu
from jax.experimental.pallas import tpu_sc as plsc

VOCAB = 1000
VOCAB_P = 1008
B_TOT = 1024 * 50
NC, NS = 2, 16
NW = NC * NS
BPW = B_TOT // NW
CH = 40
NBUF = 3
NCH = BPW // CH


@functools.partial(
    pl.kernel,
    out_type=jax.ShapeDtypeStruct((B_TOT, VOCAB), jnp.float32),
    mesh=plsc.VectorSubcoreMesh(core_axis_name="c", subcore_axis_name="s"),
    scratch_types=[
        pltpu.VMEM((BPW,), jnp.int32),
        pltpu.VMEM((NBUF, CH, VOCAB_P), jnp.float32),
        pltpu.SemaphoreType.DMA,
        pltpu.SemaphoreType.DMA,
    ],
    compiler_params=pltpu.CompilerParams(use_tc_tiling_on_sc=False),
)
def _sc_gather(idx_hbm, table_hbm, out_hbm, idx_v, rows_v, sem_g, sem_s):
    wid = lax.axis_index("s") * NC + lax.axis_index("c")
    base = wid * BPW
    pltpu.sync_copy(idx_hbm.at[pl.ds(base, BPW)], idx_v)

    gathers = [None] * NCH
    stores = [None] * NCH
    for c in range(NCH):
        if c >= NBUF:
            stores[c - NBUF].wait()
        gathers[c] = pltpu.async_copy(
            table_hbm.at[idx_v.at[pl.ds(c * CH, CH)]], rows_v.at[c % NBUF], sem_g
        )
        if c >= 1:
            gathers[c - 1].wait()
            stores[c - 1] = pltpu.async_copy(
                rows_v.at[(c - 1) % NBUF, :, pl.ds(0, VOCAB)],
                out_hbm.at[pl.ds(base + (c - 1) * CH, CH)],
                sem_s,
            )
    gathers[NCH - 1].wait()
    stores[NCH - 1] = pltpu.async_copy(
        rows_v.at[(NCH - 1) % NBUF, :, pl.ds(0, VOCAB)],
        out_hbm.at[pl.ds(base + (NCH - 1) * CH, CH)],
        sem_s,
    )
    for c in range(NCH - NBUF, NCH):
        stores[c].wait()


def kernel(idx, table):
    b, s = idx.shape
    idx_flat = idx.reshape(-1).astype(jnp.int32)
    table_p = jnp.pad(table, ((0, 0), (0, VOCAB_P - VOCAB)))
    out = _sc_gather(idx_flat, table_p)
    return out.reshape(b, s, VOCAB)


# speedup vs baseline: 1.1096x; 1.1090x over previous
import functools

import jax
import jax.numpy as jnp
from jax import lax
from jax.experimental import pallas as pl
from jax.experimental.pallas import tpu as pltpu
from jax.experimental.pallas import tpu_sc as plsc

VOCAB = 1000
B_TOT = 1024 * 50
NC, NS = 2, 16
NW = NC * NS
BPW = B_TOT // NW
CH = 32
NBUF = 2
NCH = BPW // CH
STG = VOCAB // 8


@functools.partial(
    pl.kernel,
    out_type=jax.ShapeDtypeStruct((B_TOT, VOCAB), jnp.float32),
    mesh=plsc.VectorSubcoreMesh(core_axis_name="c", subcore_axis_name="s"),
    scratch_types=[
        pltpu.VMEM((BPW,), jnp.int32),
        pltpu.VMEM((NBUF, CH, VOCAB), jnp.float32),
        pltpu.VMEM_SHARED((VOCAB, VOCAB), jnp.float32),
        pltpu.SemaphoreType.DMA,
        pltpu.SemaphoreType.DMA,
    ],
    compiler_params=pltpu.CompilerParams(use_tc_tiling_on_sc=False),
)
def _sc_gather(idx_hbm, table_hbm, out_hbm, idx_v, rows_v, table_s, sem_g, sem_s):
    sid = lax.axis_index("s")
    wid = sid * NC + lax.axis_index("c")
    base = wid * BPW
    pltpu.sync_copy(idx_hbm.at[pl.ds(base, BPW)], idx_v)

    @pl.when(sid < 8)
    def _stage():
        pltpu.sync_copy(
            table_hbm.at[pl.ds(sid * STG, STG)], table_s.at[pl.ds(sid * STG, STG)]
        )

    plsc.subcore_barrier()

    gathers = [None] * NCH
    stores = [None] * NCH
    for c in range(NCH):
        if c >= NBUF:
            stores[c - NBUF].wait()
        gathers[c] = pltpu.async_copy(
            table_s.at[idx_v.at[pl.ds(c * CH, CH)]], rows_v.at[c % NBUF], sem_g
        )
        if c >= 1:
            gathers[c - 1].wait()
            stores[c - 1] = pltpu.async_copy(
                rows_v.at[(c - 1) % NBUF],
                out_hbm.at[pl.ds(base + (c - 1) * CH, CH)],
                sem_s,
            )
    gathers[NCH - 1].wait()
    stores[NCH - 1] = pltpu.async_copy(
        rows_v.at[(NCH - 1) % NBUF],
        out_hbm.at[pl.ds(base + (NCH - 1) * CH, CH)],
        sem_s,
    )
    for c in range(NCH - NBUF, NCH):
        stores[c].wait()


def kernel(idx, table):
    b, s = idx.shape
    idx_flat = idx.reshape(-1).astype(jnp.int32)
    out = _sc_gather(idx_flat, table)
    return out.reshape(b, s, VOCAB)
